# baseline (speedup 1.0000x reference)
import functools

import jax
import jax.numpy as jnp
from jax import lax
from jax.experimental import pallas as pl
from jax.experimental.pallas import tpu as pltpu
from jax.experimental.pallas import tpu_sc as plsc

N = 10000
FEAT = 128
CTX = 288
B = 4096
E = 320000

NCORE = 2
NSUB = 16
NW = NCORE * NSUB
K = 128
NB = 80
NBC = 4
NB0 = 140
NB1 = 20
E_PAD = NSUB * (NB0 + NB1) * K
N_ACC = 10240
RPW = N_ACC // NSUB

ROWB = 400
HEADB = 512


def _mesh():
    return plsc.VectorSubcoreMesh(core_axis_name="c", subcore_axis_name="s")


@functools.partial(
    pl.kernel,
    out_type=jax.ShapeDtypeStruct((NCORE, N_ACC, 16), jnp.float32),
    mesh=_mesh(),
    compiler_params=pltpu.CompilerParams(use_tc_tiling_on_sc=False),
    scratch_types=[
        pltpu.VMEM((NBC, K), jnp.int32),
        pltpu.VMEM((K, 16), jnp.float32),
        pltpu.VMEM_SHARED((N_ACC, 16), jnp.float32),
    ],
)
def _sc_degree(dstA, dstB, zeros16, out, dstv, onesb, acc):
    c = lax.axis_index("c")
    s = lax.axis_index("s")
    for r in range(K):
        onesb[r] = jnp.ones((16,), jnp.float32)
    pltpu.sync_copy(zeros16.at[pl.ds(s * RPW, RPW)], acc.at[pl.ds(s * RPW, RPW)])
    plsc.subcore_barrier()

    def run(dsti, nb):
        for ch in range(nb // NBC):
            pltpu.sync_copy(dsti.at[s, pl.ds(ch * NBC, NBC)], dstv)

            @pl.loop(0, NBC)
            def _(g):
                pltpu.sync_copy(onesb, acc.at[dstv.at[g]], add=True)

    @pl.when(c == 0)
    def _():
        run(dstA, NB0)

    @pl.when(c == 1)
    def _():
        run(dstB, NB1)

    plsc.subcore_barrier()
    pltpu.sync_copy(acc.at[pl.ds(s * RPW, RPW)], out.at[c, pl.ds(s * RPW, RPW)])


def _make_sc_agg(nj):
    @functools.partial(
        pl.kernel,
        out_type=jax.ShapeDtypeStruct((NCORE, nj, N_ACC, FEAT), jnp.float32),
        mesh=_mesh(),
        compiler_params=pltpu.CompilerParams(use_tc_tiling_on_sc=False),
        scratch_types=[
            pltpu.VMEM((NBC, K), jnp.int32),
            pltpu.VMEM((NBC, K), jnp.int32),
            pltpu.VMEM((K, FEAT), jnp.float32),
            pltpu.VMEM((K, FEAT), jnp.float32),
            pltpu.VMEM_SHARED((N_ACC, FEAT), jnp.float32),
            pltpu.SemaphoreType.DMA,
            pltpu.SemaphoreType.DMA,
        ],
    )
    def agg(table, srcA, dstA, srcB, dstB, zeros, out,
            srcv, dstv, bufa, bufb, acc, sema, semb):
        c = lax.axis_index("c")
        s = lax.axis_index("s")

        def run(tbl, srci, dsti, nb):
            for ch in range(nb // NBC):
                pltpu.sync_copy(srci.at[s, pl.ds(ch * NBC, NBC)], srcv)
                pltpu.sync_copy(dsti.at[s, pl.ds(ch * NBC, NBC)], dstv)
                pltpu.async_copy(tbl.at[srcv.at[0]], bufa, sema)

                @pl.loop(0, NBC, step=2)
                def _(g):
                    pltpu.async_copy(tbl.at[srcv.at[g + 1]], bufb, semb)
                    pltpu.make_async_copy(tbl.at[pl.ds(0, K)], bufa, sema).wait()
                    pltpu.sync_copy(bufa, acc.at[dstv.at[g]], add=True)

                    @pl.when(g + 2 < NBC)
                    def _():
                        pltpu.async_copy(tbl.at[srcv.at[g + 2]], bufa, sema)

                    pltpu.make_async_copy(tbl.at[pl.ds(0, K)], bufb, semb).wait()
                    pltpu.sync_copy(bufb, acc.at[dstv.at[g + 1]], add=True)

        for j in range(nj):
            tbl = table.at[j]
            pltpu.sync_copy(zeros.at[pl.ds(s * RPW, RPW)], acc.at[pl.ds(s * RPW, RPW)])
            plsc.subcore_barrier()

            @pl.when(c == 0)
            def _():
                run(tbl, srcA, dstA, NB0)

            @pl.when(c == 1)
            def _():
                run(tbl, srcB, dstB, NB1)

            plsc.subcore_barrier()
            pltpu.sync_copy(
                acc.at[pl.ds(s * RPW, RPW)], out.at[c, j, pl.ds(s * RPW, RPW)]
            )
            if j + 1 < nj:
                plsc.subcore_barrier()

    return agg


_sc_agg1 = _make_sc_agg(1)
_sc_agg4 = _make_sc_agg(4)


@functools.partial(
    pl.kernel,
    out_type=[
        jax.ShapeDtypeStruct((B, FEAT), jnp.float32),
        jax.ShapeDtypeStruct((B, FEAT), jnp.float32),
        jax.ShapeDtypeStruct((B, CTX), jnp.float32),
    ],
    mesh=_mesh(),
    compiler_params=pltpu.CompilerParams(use_tc_tiling_on_sc=False),
    scratch_types=[
        pltpu.VMEM((K,), jnp.int32),
        pltpu.VMEM((K, FEAT), jnp.float32),
        pltpu.VMEM((K, CTX), jnp.float32),
        pltpu.SemaphoreType.DMA,
    ],
)
def _sc_pair(h, ctx, i1, i2, i3, o1, o2, o3, idxv, bufh, bufc, sem):
    c = lax.axis_index("c")
    s = lax.axis_index("s")
    w = s * NCORE + c
    base = w * K
    pltpu.sync_copy(i1.at[w], idxv)
    pltpu.async_copy(h.at[idxv], bufh, sem).wait()
    pltpu.sync_copy(bufh, o1.at[pl.ds(base, K)])
    pltpu.sync_copy(i2.at[w], idxv)
    pltpu.async_copy(h.at[idxv], bufh, sem).wait()
    pltpu.sync_copy(bufh, o2.at[pl.ds(base, K)])
    pltpu.sync_copy(i3.at[w], idxv)
    pltpu.async_copy(ctx.at[idxv], bufc, sem).wait()
    pltpu.sync_copy(bufc, o3.at[pl.ds(base, K)])


def _dot(a, b):
    return jnp.dot(a, b, preferred_element_type=jnp.float32)


def _tc_prep_body(hist_ref, nf_ref, dinv_ref, xs0_ref):
    h = hist_ref[...]
    deg = h[0, :, 0] + h[1, :, 0] + 1.0
    dinv = lax.rsqrt(deg).reshape(ROWB, 1)
    dinv_ref[...] = dinv
    xs0_ref[...] = nf_ref[...] * dinv


def _tc_prep(hist, node_feature):
    grid = N // ROWB
    return pl.pallas_call(
        _tc_prep_body,
        grid=(grid,),
        in_specs=[
            pl.BlockSpec((NCORE, ROWB, 16), lambda i: (0, i, 0)),
            pl.BlockSpec((ROWB, FEAT), lambda i: (i, 0)),
        ],
        out_specs=[
            pl.BlockSpec((ROWB, 1), lambda i: (i, 0)),
            pl.BlockSpec((ROWB, FEAT), lambda i: (i, 0)),
        ],
        out_shape=[
            jax.ShapeDtypeStruct((N, 1), jnp.float32),
            jax.ShapeDtypeStruct((N, FEAT), jnp.float32),
        ],
    )(hist, node_feature)


def _tc_conv1_body(a_ref, xs_ref, dinv_ref, w0_ref, b0_ref, w1_ref, out_ref):
    dinv = dinv_ref[...]
    y = (a_ref[0] + a_ref[1] + xs_ref[...]) * dinv
    h1 = jnp.maximum(_dot(y, w0_ref[...]) + b0_ref[...], 0.0)
    zs = _dot(h1 * dinv, w1_ref[...])
    for j in range(4):
        out_ref[j] = zs[:, j * FEAT:(j + 1) * FEAT]


def _tc_conv1(agg1, xs0, dinv, Wg0, bg0, Wg1):
    grid = N // ROWB
    return pl.pallas_call(
        _tc_conv1_body,
        grid=(grid,),
        in_specs=[
            pl.BlockSpec((NCORE, ROWB, FEAT), lambda i: (0, i, 0)),
            pl.BlockSpec((ROWB, FEAT), lambda i: (i, 0)),
            pl.BlockSpec((ROWB, 1), lambda i: (i, 0)),
            pl.BlockSpec((FEAT, 1024), lambda i: (0, 0)),
            pl.BlockSpec((1, 1024), lambda i: (0, 0)),
            pl.BlockSpec((1024, 512), lambda i: (0, 0)),
        ],
        out_specs=pl.BlockSpec((4, ROWB, FEAT), lambda i: (0, i, 0)),
        out_shape=jax.ShapeDtypeStruct((4, N, FEAT), jnp.float32),
    )(agg1, xs0, dinv, Wg0, bg0.reshape(1, -1), Wg1)


def _tc_conv2_body(a_ref, zs_ref, dinv_ref, b1_ref, w2_ref, out_ref):
    dinv = dinv_ref[...]
    parts = [a_ref[0, j] + a_ref[1, j] + zs_ref[j] for j in range(4)]
    y = jnp.concatenate(parts, axis=1) * dinv
    h2 = jnp.maximum(y + b1_ref[...], 0.0)
    out_ref[...] = _dot(h2 * dinv, w2_ref[...])


def _tc_conv2(agg2, zs2c, dinv, bg1, Wg2):
    grid = N // ROWB
    return pl.pallas_call(
        _tc_conv2_body,
        grid=(grid,),
        in_specs=[
            pl.BlockSpec((NCORE, 4, ROWB, FEAT), lambda i: (0, 0, i, 0)),
            pl.BlockSpec((4, ROWB, FEAT), lambda i: (0, i, 0)),
            pl.BlockSpec((ROWB, 1), lambda i: (i, 0)),
            pl.BlockSpec((1, 512), lambda i: (0, 0)),
            pl.BlockSpec((512, FEAT), lambda i: (0, 0)),
        ],
        out_specs=pl.BlockSpec((ROWB, FEAT), lambda i: (i, 0)),
        out_shape=jax.ShapeDtypeStruct((N, FEAT), jnp.float32),
    )(agg2, zs2c, dinv, bg1.reshape(1, -1), Wg2)


def _tc_conv3_body(a_ref, zs_ref, dinv_ref, b2_ref, out_ref):
    y = (a_ref[0] + a_ref[1] + zs_ref[...]) * dinv_ref[...]
    out_ref[...] = jnp.maximum(y + b2_ref[...], 0.0)


def _tc_conv3(agg3, zs3, dinv, bg2):
    grid = N // ROWB
    return pl.pallas_call(
        _tc_conv3_body,
        grid=(grid,),
        in_specs=[
            pl.BlockSpec((NCORE, ROWB, FEAT), lambda i: (0, i, 0)),
            pl.BlockSpec((ROWB, FEAT), lambda i: (i, 0)),
            pl.BlockSpec((ROWB, 1), lambda i: (i, 0)),
            pl.BlockSpec((1, FEAT), lambda i: (0, 0)),
        ],
        out_specs=pl.BlockSpec((ROWB, FEAT), lambda i: (i, 0)),
        out_shape=jax.ShapeDtypeStruct((N, FEAT), jnp.float32),
    )(agg3, zs3, dinv, bg2.reshape(1, -1))


def _tc_head_body(x1_ref, x2_ref, x3_ref, wc0, bc0, wc1, bc1, wc2, bc2,
                  wf0, bf0, wf1, bf1, wf2, bf2, out_ref):
    t = jnp.maximum(_dot(x3_ref[...], wc0[...]) + bc0[...], 0.0)
    t = jnp.maximum(_dot(t, wc1[...]) + bc1[...], 0.0)
    t3 = _dot(t, wc2[...]) + bc2[...]
    x = jnp.concatenate([x1_ref[...], x2_ref[...], t3], axis=1)
    z = jnp.maximum(_dot(x, wf0[...]) + bf0[...], 0.0)
    z = jnp.maximum(_dot(z, wf1[...]) + bf1[...], 0.0)
    out_ref[...] = _dot(z, wf2[...]) + bf2[...]


def _tc_head(x1, x2, x3, Wc0, bc0, Wc1, bc1, Wc2, bc2,
             Wf0, bf0, Wf1, bf1, Wf2, bf2):
    grid = B // HEADB

    def full(shape):
        return pl.BlockSpec(shape, lambda i: tuple(0 for _ in shape))

    return pl.pallas_call(
        _tc_head_body,
        grid=(grid,),
        in_specs=[
            pl.BlockSpec((HEADB, FEAT), lambda i: (i, 0)),
            pl.BlockSpec((HEADB, FEAT), lambda i: (i, 0)),
            pl.BlockSpec((HEADB, CTX), lambda i: (i, 0)),
            full((CTX, 2048)), full((1, 2048)),
            full((2048, 512)), full((1, 512)),
            full((512, FEAT)), full((1, FEAT)),
            full((384, FEAT)), full((1, FEAT)),
            full((FEAT, 64)), full((1, 64)),
            full((64, 1)), full((1, 1)),
        ],
        out_specs=pl.BlockSpec((HEADB, 1), lambda i: (i, 0)),
        out_shape=jax.ShapeDtypeStruct((B, 1), jnp.float32),
    )(x1, x2, x3, Wc0, bc0.reshape(1, -1), Wc1, bc1.reshape(1, -1),
      Wc2, bc2.reshape(1, -1), Wf0, bf0.reshape(1, -1), Wf1, bf1.reshape(1, -1),
      Wf2, bf2.reshape(1, -1))


def kernel(inputs, node_feature, edge_index, Wg0, bg0, Wg1, bg1, Wg2, bg2,
           context_table, Wc0, bc0, Wc1, bc1, Wc2, bc2,
           Wf0, bf0, Wf1, bf1, Wf2, bf2):
    pad = E_PAD - E
    src = jnp.concatenate([edge_index[0], jnp.zeros((pad,), jnp.int32)])
    dst = jnp.concatenate([edge_index[1], jnp.full((pad,), N, jnp.int32)])
    ea = NSUB * NB0 * K
    srcA = src[:ea].reshape(NSUB, NB0, K)
    dstA = dst[:ea].reshape(NSUB, NB0, K)
    srcB = src[ea:].reshape(NSUB, NB1, K)
    dstB = dst[ea:].reshape(NSUB, NB1, K)
    zeros = jnp.zeros((N_ACC, FEAT), jnp.float32)
    zeros16 = jnp.zeros((N_ACC, 16), jnp.float32)

    hist = _sc_degree(dstA, dstB, zeros16)
    dinv, xs0 = _tc_prep(hist, node_feature)

    agg1 = _sc_agg1(xs0.reshape(1, N, FEAT), srcA, dstA, srcB, dstB, zeros)
    zs2c = _tc_conv1(agg1.reshape(NCORE, N_ACC, FEAT), xs0, dinv, Wg0, bg0, Wg1)

    agg2 = _sc_agg4(zs2c, srcA, dstA, srcB, dstB, zeros)
    zs3 = _tc_conv2(agg2, zs2c, dinv, bg1, Wg2)

    agg3 = _sc_agg1(zs3.reshape(1, N, FEAT), srcA, dstA, srcB, dstB, zeros)
    h = _tc_conv3(agg3.reshape(NCORE, N_ACC, FEAT), zs3, dinv, bg2)

    i1 = inputs[:, 0].reshape(NW, K)
    i2 = inputs[:, 1].reshape(NW, K)
    i3 = inputs[:, 2].reshape(NW, K)
    x1, x2, x3 = _sc_pair(h, context_table, i1, i2, i3)

    return _tc_head(x1, x2, x3, Wc0, bc0, Wc1, bc1, Wc2, bc2,
                    Wf0, bf0, Wf1, bf1, Wf2, bf2)


# speedup vs baseline: 1.7821x; 1.7821x over previous
import functools

import jax
import jax.numpy as jnp
from jax import lax
from jax.experimental import pallas as pl
from jax.experimental.pallas import tpu as pltpu
from jax.experimental.pallas import tpu_sc as plsc

N = 10000
FEAT = 128
CTX = 288
B = 4096
E = 320000

NCORE = 2
NSUB = 16
NW = NCORE * NSUB
K = 128
NBC = 8
NB0 = 80
NB1 = 80
E_PAD = NSUB * (NB0 + NB1) * K
N_ACC = 10240
RPW = N_ACC // NSUB
HW = 64
N_ACC2 = 10016
RPW2 = N_ACC2 // NSUB
TRW = N // NSUB

ROWB = 400
HEADB = 512


def _mesh():
    return plsc.VectorSubcoreMesh(core_axis_name="c", subcore_axis_name="s")


@functools.partial(
    pl.kernel,
    out_type=jax.ShapeDtypeStruct((NCORE, N_ACC, 16), jnp.float32),
    mesh=_mesh(),
    compiler_params=pltpu.CompilerParams(use_tc_tiling_on_sc=False),
    scratch_types=[
        pltpu.VMEM((NBC, K), jnp.int32),
        pltpu.VMEM((K, 16), jnp.float32),
        pltpu.VMEM_SHARED((N_ACC, 16), jnp.float32),
    ],
)
def _sc_degree(dstA, dstB, zeros16, out, dstv, onesb, acc):
    c = lax.axis_index("c")
    s = lax.axis_index("s")
    for r in range(K):
        onesb[r] = jnp.ones((16,), jnp.float32)
    pltpu.sync_copy(zeros16.at[pl.ds(s * RPW, RPW)], acc.at[pl.ds(s * RPW, RPW)])
    plsc.subcore_barrier()

    def run(dsti, nb):
        @pl.loop(0, nb // NBC)
        def _(ch):
            pltpu.sync_copy(dsti.at[s, pl.ds(ch * NBC, NBC)], dstv)

            @pl.loop(0, NBC)
            def _(g):
                pltpu.sync_copy(onesb, acc.at[dstv.at[g]], add=True)

    @pl.when(c == 0)
    def _():
        run(dstA, NB0)

    @pl.when(c == 1)
    def _():
        run(dstB, NB1)

    plsc.subcore_barrier()
    pltpu.sync_copy(acc.at[pl.ds(s * RPW, RPW)], out.at[c, pl.ds(s * RPW, RPW)])


def _make_sc_agg(nj):
    @functools.partial(
        pl.kernel,
        out_type=jax.ShapeDtypeStruct((NCORE, nj, N_ACC2, FEAT), jnp.float32),
        mesh=_mesh(),
        compiler_params=pltpu.CompilerParams(use_tc_tiling_on_sc=False),
        scratch_types=[
            pltpu.VMEM((NBC, K), jnp.int32),
            pltpu.VMEM((NBC, K), jnp.int32),
            pltpu.VMEM((K, HW), jnp.float32),
            pltpu.VMEM((K, HW), jnp.float32),
            pltpu.VMEM_SHARED((N, HW), jnp.float32),
            pltpu.VMEM_SHARED((N_ACC2, HW), jnp.float32),
            pltpu.SemaphoreType.DMA,
            pltpu.SemaphoreType.DMA,
        ],
    )
    def agg(table, srcA, dstA, srcB, dstB, zeros, out,
            srcv, dstv, bufa, bufb, tbl, acc, sema, semb):
        c = lax.axis_index("c")
        s = lax.axis_index("s")

        def run(srci, dsti, nb):
            @pl.loop(0, nb // NBC)
            def _(ch):
                pltpu.sync_copy(srci.at[s, pl.ds(ch * NBC, NBC)], srcv)
                pltpu.sync_copy(dsti.at[s, pl.ds(ch * NBC, NBC)], dstv)
                pltpu.async_copy(tbl.at[srcv.at[0]], bufa, sema)

                @pl.loop(0, NBC, step=2)
                def _(g):
                    pltpu.async_copy(tbl.at[srcv.at[g + 1]], bufb, semb)
                    pltpu.make_async_copy(tbl.at[pl.ds(0, K)], bufa, sema).wait()
                    pltpu.sync_copy(bufa, acc.at[dstv.at[g]], add=True)

                    @pl.when(g + 2 < NBC)
                    def _():
                        pltpu.async_copy(tbl.at[srcv.at[g + 2]], bufa, sema)

                    pltpu.make_async_copy(tbl.at[pl.ds(0, K)], bufb, semb).wait()
                    pltpu.sync_copy(bufb, acc.at[dstv.at[g + 1]], add=True)

        for j in range(nj):
            for h in range(2):
                pltpu.sync_copy(
                    table.at[j, pl.ds(s * TRW, TRW), pl.ds(h * HW, HW)],
                    tbl.at[pl.ds(s * TRW, TRW)],
                )
                pltpu.sync_copy(
                    zeros.at[pl.ds(s * RPW2, RPW2)], acc.at[pl.ds(s * RPW2, RPW2)]
                )
                plsc.subcore_barrier()

                @pl.when(c == 0)
                def _():
                    run(srcA, dstA, NB0)

                @pl.when(c == 1)
                def _():
                    run(srcB, dstB, NB1)

                plsc.subcore_barrier()
                pltpu.sync_copy(
                    acc.at[pl.ds(s * RPW2, RPW2)],
                    out.at[c, j, pl.ds(s * RPW2, RPW2), pl.ds(h * HW, HW)],
                )
                if j + 1 < nj or h == 0:
                    plsc.subcore_barrier()

    return agg


_sc_agg1 = _make_sc_agg(1)
_sc_agg4 = _make_sc_agg(4)


@functools.partial(
    pl.kernel,
    out_type=[
        jax.ShapeDtypeStruct((B, FEAT), jnp.float32),
        jax.ShapeDtypeStruct((B, FEAT), jnp.float32),
        jax.ShapeDtypeStruct((B, CTX), jnp.float32),
    ],
    mesh=_mesh(),
    compiler_params=pltpu.CompilerParams(use_tc_tiling_on_sc=False),
    scratch_types=[
        pltpu.VMEM((K,), jnp.int32),
        pltpu.VMEM((K, FEAT), jnp.float32),
        pltpu.VMEM((K, CTX), jnp.float32),
        pltpu.SemaphoreType.DMA,
    ],
)
def _sc_pair(h, ctx, i1, i2, i3, o1, o2, o3, idxv, bufh, bufc, sem):
    c = lax.axis_index("c")
    s = lax.axis_index("s")
    w = s * NCORE + c
    base = w * K
    pltpu.sync_copy(i1.at[w], idxv)
    pltpu.async_copy(h.at[idxv], bufh, sem).wait()
    pltpu.sync_copy(bufh, o1.at[pl.ds(base, K)])
    pltpu.sync_copy(i2.at[w], idxv)
    pltpu.async_copy(h.at[idxv], bufh, sem).wait()
    pltpu.sync_copy(bufh, o2.at[pl.ds(base, K)])
    pltpu.sync_copy(i3.at[w], idxv)
    pltpu.async_copy(ctx.at[idxv], bufc, sem).wait()
    pltpu.sync_copy(bufc, o3.at[pl.ds(base, K)])


def _dot(a, b):
    return jnp.dot(a, b, preferred_element_type=jnp.float32)


def _tc_prep_body(hist_ref, nf_ref, dinv_ref, xs0_ref):
    h = hist_ref[...]
    deg = h[0, :, 0] + h[1, :, 0] + 1.0
    dinv = lax.rsqrt(deg).reshape(ROWB, 1)
    dinv_ref[...] = dinv
    xs0_ref[...] = nf_ref[...] * dinv


def _tc_prep(hist, node_feature):
    grid = N // ROWB
    return pl.pallas_call(
        _tc_prep_body,
        grid=(grid,),
        in_specs=[
            pl.BlockSpec((NCORE, ROWB, 16), lambda i: (0, i, 0)),
            pl.BlockSpec((ROWB, FEAT), lambda i: (i, 0)),
        ],
        out_specs=[
            pl.BlockSpec((ROWB, 1), lambda i: (i, 0)),
            pl.BlockSpec((ROWB, FEAT), lambda i: (i, 0)),
        ],
        out_shape=[
            jax.ShapeDtypeStruct((N, 1), jnp.float32),
            jax.ShapeDtypeStruct((N, FEAT), jnp.float32),
        ],
    )(hist, node_feature)


def _tc_conv1_body(a_ref, xs_ref, dinv_ref, w0_ref, b0_ref, w1_ref, out_ref):
    dinv = dinv_ref[...]
    y = (a_ref[0, 0] + a_ref[1, 0] + xs_ref[...]) * dinv
    h1 = jnp.maximum(_dot(y, w0_ref[...]) + b0_ref[...], 0.0)
    zs = _dot(h1 * dinv, w1_ref[...])
    for j in range(4):
        out_ref[j] = zs[:, j * FEAT:(j + 1) * FEAT]


def _tc_conv1(agg1, xs0, dinv, Wg0, bg0, Wg1):
    grid = N // ROWB
    return pl.pallas_call(
        _tc_conv1_body,
        grid=(grid,),
        in_specs=[
            pl.BlockSpec((NCORE, 1, ROWB, FEAT), lambda i: (0, 0, i, 0)),
            pl.BlockSpec((ROWB, FEAT), lambda i: (i, 0)),
            pl.BlockSpec((ROWB, 1), lambda i: (i, 0)),
            pl.BlockSpec((FEAT, 1024), lambda i: (0, 0)),
            pl.BlockSpec((1, 1024), lambda i: (0, 0)),
            pl.BlockSpec((1024, 512), lambda i: (0, 0)),
        ],
        out_specs=pl.BlockSpec((4, ROWB, FEAT), lambda i: (0, i, 0)),
        out_shape=jax.ShapeDtypeStruct((4, N, FEAT), jnp.float32),
    )(agg1, xs0, dinv, Wg0, bg0.reshape(1, -1), Wg1)


def _tc_conv2_body(a_ref, zs_ref, dinv_ref, b1_ref, w2_ref, out_ref):
    dinv = dinv_ref[...]
    parts = [a_ref[0, j] + a_ref[1, j] + zs_ref[j] for j in range(4)]
    y = jnp.concatenate(parts, axis=1) * dinv
    h2 = jnp.maximum(y + b1_ref[...], 0.0)
    out_ref[...] = _dot(h2 * dinv, w2_ref[...])


def _tc_conv2(agg2, zs2c, dinv, bg1, Wg2):
    grid = N // ROWB
    return pl.pallas_call(
        _tc_conv2_body,
        grid=(grid,),
        in_specs=[
            pl.BlockSpec((NCORE, 4, ROWB, FEAT), lambda i: (0, 0, i, 0)),
            pl.BlockSpec((4, ROWB, FEAT), lambda i: (0, i, 0)),
            pl.BlockSpec((ROWB, 1), lambda i: (i, 0)),
            pl.BlockSpec((1, 512), lambda i: (0, 0)),
            pl.BlockSpec((512, FEAT), lambda i: (0, 0)),
        ],
        out_specs=pl.BlockSpec((ROWB, FEAT), lambda i: (i, 0)),
        out_shape=jax.ShapeDtypeStruct((N, FEAT), jnp.float32),
    )(agg2, zs2c, dinv, bg1.reshape(1, -1), Wg2)


def _tc_conv3_body(a_ref, zs_ref, dinv_ref, b2_ref, out_ref):
    y = (a_ref[0, 0] + a_ref[1, 0] + zs_ref[...]) * dinv_ref[...]
    out_ref[...] = jnp.maximum(y + b2_ref[...], 0.0)


def _tc_conv3(agg3, zs3, dinv, bg2):
    grid = N // ROWB
    return pl.pallas_call(
        _tc_conv3_body,
        grid=(grid,),
        in_specs=[
            pl.BlockSpec((NCORE, 1, ROWB, FEAT), lambda i: (0, 0, i, 0)),
            pl.BlockSpec((ROWB, FEAT), lambda i: (i, 0)),
            pl.BlockSpec((ROWB, 1), lambda i: (i, 0)),
            pl.BlockSpec((1, FEAT), lambda i: (0, 0)),
        ],
        out_specs=pl.BlockSpec((ROWB, FEAT), lambda i: (i, 0)),
        out_shape=jax.ShapeDtypeStruct((N, FEAT), jnp.float32),
    )(agg3, zs3, dinv, bg2.reshape(1, -1))


def _tc_head_body(x1_ref, x2_ref, x3_ref, wc0, bc0, wc1, bc1, wc2, bc2,
                  wf0, bf0, wf1, bf1, wf2, bf2, out_ref):
    t = jnp.maximum(_dot(x3_ref[...], wc0[...]) + bc0[...], 0.0)
    t = jnp.maximum(_dot(t, wc1[...]) + bc1[...], 0.0)
    t3 = _dot(t, wc2[...]) + bc2[...]
    x = jnp.concatenate([x1_ref[...], x2_ref[...], t3], axis=1)
    z = jnp.maximum(_dot(x, wf0[...]) + bf0[...], 0.0)
    z = jnp.maximum(_dot(z, wf1[...]) + bf1[...], 0.0)
    out_ref[...] = _dot(z, wf2[...]) + bf2[...]


def _tc_head(x1, x2, x3, Wc0, bc0, Wc1, bc1, Wc2, bc2,
             Wf0, bf0, Wf1, bf1, Wf2, bf2):
    grid = B // HEADB

    def full(shape):
        return pl.BlockSpec(shape, lambda i: tuple(0 for _ in shape))

    return pl.pallas_call(
        _tc_head_body,
        grid=(grid,),
        in_specs=[
            pl.BlockSpec((HEADB, FEAT), lambda i: (i, 0)),
            pl.BlockSpec((HEADB, FEAT), lambda i: (i, 0)),
            pl.BlockSpec((HEADB, CTX), lambda i: (i, 0)),
            full((CTX, 2048)), full((1, 2048)),
            full((2048, 512)), full((1, 512)),
            full((512, FEAT)), full((1, FEAT)),
            full((384, FEAT)), full((1, FEAT)),
            full((FEAT, 64)), full((1, 64)),
            full((64, 1)), full((1, 1)),
        ],
        out_specs=pl.BlockSpec((HEADB, 1), lambda i: (i, 0)),
        out_shape=jax.ShapeDtypeStruct((B, 1), jnp.float32),
    )(x1, x2, x3, Wc0, bc0.reshape(1, -1), Wc1, bc1.reshape(1, -1),
      Wc2, bc2.reshape(1, -1), Wf0, bf0.reshape(1, -1), Wf1, bf1.reshape(1, -1),
      Wf2, bf2.reshape(1, -1))


def kernel(inputs, node_feature, edge_index, Wg0, bg0, Wg1, bg1, Wg2, bg2,
           context_table, Wc0, bc0, Wc1, bc1, Wc2, bc2,
           Wf0, bf0, Wf1, bf1, Wf2, bf2):
    pad = E_PAD - E
    src = jnp.concatenate([edge_index[0], jnp.zeros((pad,), jnp.int32)])
    dst = jnp.concatenate([edge_index[1], jnp.full((pad,), N, jnp.int32)])
    ea = NSUB * NB0 * K
    srcA = src[:ea].reshape(NSUB, NB0, K)
    dstA = dst[:ea].reshape(NSUB, NB0, K)
    srcB = src[ea:].reshape(NSUB, NB1, K)
    dstB = dst[ea:].reshape(NSUB, NB1, K)
    zeros = jnp.zeros((N_ACC2, HW), jnp.float32)
    zeros16 = jnp.zeros((N_ACC, 16), jnp.float32)

    hist = _sc_degree(dstA, dstB, zeros16)
    dinv, xs0 = _tc_prep(hist, node_feature)

    agg1 = _sc_agg1(xs0.reshape(1, N, FEAT), srcA, dstA, srcB, dstB, zeros)
    zs2c = _tc_conv1(agg1, xs0, dinv, Wg0, bg0, Wg1)

    agg2 = _sc_agg4(zs2c, srcA, dstA, srcB, dstB, zeros)
    zs3 = _tc_conv2(agg2, zs2c, dinv, bg1, Wg2)

    agg3 = _sc_agg1(zs3.reshape(1, N, FEAT), srcA, dstA, srcB, dstB, zeros)
    h = _tc_conv3(agg3, zs3, dinv, bg2)

    i1 = inputs[:, 0].reshape(NW, K)
    i2 = inputs[:, 1].reshape(NW, K)
    i3 = inputs[:, 2].reshape(NW, K)
    x1, x2, x3 = _sc_pair(h, context_table, i1, i2, i3)

    return _tc_head(x1, x2, x3, Wc0, bc0, Wc1, bc1, Wc2, bc2,
                    Wf0, bf0, Wf1, bf1, Wf2, bf2)


# speedup vs baseline: 2.2408x; 1.2574x over previous
import functools

import jax
import jax.numpy as jnp
from jax import lax
from jax.experimental import pallas as pl
from jax.experimental.pallas import tpu as pltpu
from jax.experimental.pallas import tpu_sc as plsc

N = 10000
FEAT = 128
CTX = 288
B = 4096
E = 320000

NCORE = 2
NSUB = 16
NW = NCORE * NSUB
K = 128
NBC = 16
NB0 = 80
NB1 = 80
E_PAD = NSUB * (NB0 + NB1) * K
N_ACC = 10240
RPW = N_ACC // NSUB
HW = 64
N_ACC2 = 10016
RPW2 = N_ACC2 // NSUB
TRW = N // NSUB

ROWB = 400
HEADB = 512


def _mesh():
    return plsc.VectorSubcoreMesh(core_axis_name="c", subcore_axis_name="s")


@functools.partial(
    pl.kernel,
    out_type=jax.ShapeDtypeStruct((NCORE, N_ACC, 16), jnp.float32),
    mesh=_mesh(),
    compiler_params=pltpu.CompilerParams(use_tc_tiling_on_sc=False),
    scratch_types=[
        pltpu.VMEM((NBC, K), jnp.int32),
        pltpu.VMEM((K, 16), jnp.float32),
        pltpu.VMEM_SHARED((N_ACC, 16), jnp.float32),
    ],
)
def _sc_degree(dstA, dstB, zeros16, out, dstv, onesb, acc):
    c = lax.axis_index("c")
    s = lax.axis_index("s")
    for r in range(K):
        onesb[r] = jnp.ones((16,), jnp.float32)
    pltpu.sync_copy(zeros16.at[pl.ds(s * RPW, RPW)], acc.at[pl.ds(s * RPW, RPW)])
    plsc.subcore_barrier()

    def run(dsti, nb):
        @pl.loop(0, nb // NBC)
        def _(ch):
            pltpu.sync_copy(dsti.at[s, pl.ds(ch * NBC, NBC)], dstv)

            @pl.loop(0, NBC)
            def _(g):
                pltpu.sync_copy(onesb, acc.at[dstv.at[g]], add=True)

    @pl.when(c == 0)
    def _():
        run(dstA, NB0)

    @pl.when(c == 1)
    def _():
        run(dstB, NB1)

    plsc.subcore_barrier()
    pltpu.sync_copy(acc.at[pl.ds(s * RPW, RPW)], out.at[c, pl.ds(s * RPW, RPW)])


def _make_sc_agg(nj):
    @functools.partial(
        pl.kernel,
        out_type=jax.ShapeDtypeStruct((NCORE, nj, N_ACC2, FEAT), jnp.float32),
        mesh=_mesh(),
        compiler_params=pltpu.CompilerParams(use_tc_tiling_on_sc=False),
        scratch_types=[
            pltpu.VMEM((NBC, K), jnp.int32),
            pltpu.VMEM((NBC, K), jnp.int32),
            pltpu.VMEM((K, HW), jnp.float32),
            pltpu.VMEM((K, HW), jnp.float32),
            pltpu.VMEM((K, HW), jnp.float32),
            pltpu.VMEM((K, HW), jnp.float32),
            pltpu.VMEM_SHARED((N, HW), jnp.float32),
            pltpu.VMEM_SHARED((N_ACC2, HW), jnp.float32),
            pltpu.SemaphoreType.DMA,
            pltpu.SemaphoreType.DMA,
        ],
    )
    def agg(table, srcA, dstA, srcB, dstB, zeros, out,
            srcv, dstv, bufa, bufb, bufc, bufd, tbl, acc, semg, sems):
        c = lax.axis_index("c")
        s = lax.axis_index("s")
        bufs = (bufa, bufb, bufc, bufd)

        def wait_gather(b):
            pltpu.make_async_copy(tbl.at[pl.ds(0, K)], b, semg).wait()

        def wait_scatter():
            pltpu.make_async_copy(bufa, acc.at[dstv.at[0]], sems).wait()

        def run(srci, dsti, nb):
            @pl.loop(0, nb // NBC)
            def _(ch):
                pltpu.sync_copy(srci.at[s, pl.ds(ch * NBC, NBC)], srcv)
                pltpu.sync_copy(dsti.at[s, pl.ds(ch * NBC, NBC)], dstv)
                for g in range(3):
                    pltpu.async_copy(tbl.at[srcv.at[g]], bufs[g], semg)
                for g in range(NBC):
                    b = bufs[g % 4]
                    wait_gather(b)
                    pltpu.async_copy(b, acc.at[dstv.at[g]], sems, add=True)
                    if g + 3 < NBC:
                        if g >= 1:
                            wait_scatter()
                        pltpu.async_copy(tbl.at[srcv.at[g + 3]], bufs[(g + 3) % 4], semg)
                for _ in range(min(4, NBC)):
                    wait_scatter()

        for j in range(nj):
            for h in range(2):
                pltpu.sync_copy(
                    table.at[j, pl.ds(s * TRW, TRW), pl.ds(h * HW, HW)],
                    tbl.at[pl.ds(s * TRW, TRW)],
                )
                pltpu.sync_copy(
                    zeros.at[pl.ds(s * RPW2, RPW2)], acc.at[pl.ds(s * RPW2, RPW2)]
                )
                plsc.subcore_barrier()

                @pl.when(c == 0)
                def _():
                    run(srcA, dstA, NB0)

                @pl.when(c == 1)
                def _():
                    run(srcB, dstB, NB1)

                plsc.subcore_barrier()
                pltpu.sync_copy(
                    acc.at[pl.ds(s * RPW2, RPW2)],
                    out.at[c, j, pl.ds(s * RPW2, RPW2), pl.ds(h * HW, HW)],
                )
                if j + 1 < nj or h == 0:
                    plsc.subcore_barrier()

    return agg


_sc_agg1 = _make_sc_agg(1)
_sc_agg4 = _make_sc_agg(4)


@functools.partial(
    pl.kernel,
    out_type=[
        jax.ShapeDtypeStruct((B, FEAT), jnp.float32),
        jax.ShapeDtypeStruct((B, FEAT), jnp.float32),
        jax.ShapeDtypeStruct((B, CTX), jnp.float32),
    ],
    mesh=_mesh(),
    compiler_params=pltpu.CompilerParams(use_tc_tiling_on_sc=False),
    scratch_types=[
        pltpu.VMEM((K,), jnp.int32),
        pltpu.VMEM((K, FEAT), jnp.float32),
        pltpu.VMEM((K, CTX), jnp.float32),
        pltpu.SemaphoreType.DMA,
    ],
)
def _sc_pair(h, ctx, i1, i2, i3, o1, o2, o3, idxv, bufh, bufc, sem):
    c = lax.axis_index("c")
    s = lax.axis_index("s")
    w = s * NCORE + c
    base = w * K
    pltpu.sync_copy(i1.at[w], idxv)
    pltpu.async_copy(h.at[idxv], bufh, sem).wait()
    pltpu.sync_copy(bufh, o1.at[pl.ds(base, K)])
    pltpu.sync_copy(i2.at[w], idxv)
    pltpu.async_copy(h.at[idxv], bufh, sem).wait()
    pltpu.sync_copy(bufh, o2.at[pl.ds(base, K)])
    pltpu.sync_copy(i3.at[w], idxv)
    pltpu.async_copy(ctx.at[idxv], bufc, sem).wait()
    pltpu.sync_copy(bufc, o3.at[pl.ds(base, K)])


def _dot(a, b):
    return jnp.dot(a, b, preferred_element_type=jnp.float32)


def _tc_prep_body(hist_ref, nf_ref, dinv_ref, xs0_ref):
    h = hist_ref[...]
    deg = h[0, :, 0] + h[1, :, 0] + 1.0
    dinv = lax.rsqrt(deg).reshape(ROWB, 1)
    dinv_ref[...] = dinv
    xs0_ref[...] = nf_ref[...] * dinv


def _tc_prep(hist, node_feature):
    grid = N // ROWB
    return pl.pallas_call(
        _tc_prep_body,
        grid=(grid,),
        in_specs=[
            pl.BlockSpec((NCORE, ROWB, 16), lambda i: (0, i, 0)),
            pl.BlockSpec((ROWB, FEAT), lambda i: (i, 0)),
        ],
        out_specs=[
            pl.BlockSpec((ROWB, 1), lambda i: (i, 0)),
            pl.BlockSpec((ROWB, FEAT), lambda i: (i, 0)),
        ],
        out_shape=[
            jax.ShapeDtypeStruct((N, 1), jnp.float32),
            jax.ShapeDtypeStruct((N, FEAT), jnp.float32),
        ],
    )(hist, node_feature)


def _tc_conv1_body(a_ref, xs_ref, dinv_ref, w0_ref, b0_ref, w1_ref, out_ref):
    dinv = dinv_ref[...]
    y = (a_ref[0, 0] + a_ref[1, 0] + xs_ref[...]) * dinv
    h1 = jnp.maximum(_dot(y, w0_ref[...]) + b0_ref[...], 0.0)
    zs = _dot(h1 * dinv, w1_ref[...])
    for j in range(4):
        out_ref[j] = zs[:, j * FEAT:(j + 1) * FEAT]


def _tc_conv1(agg1, xs0, dinv, Wg0, bg0, Wg1):
    grid = N // ROWB
    return pl.pallas_call(
        _tc_conv1_body,
        grid=(grid,),
        in_specs=[
            pl.BlockSpec((NCORE, 1, ROWB, FEAT), lambda i: (0, 0, i, 0)),
            pl.BlockSpec((ROWB, FEAT), lambda i: (i, 0)),
            pl.BlockSpec((ROWB, 1), lambda i: (i, 0)),
            pl.BlockSpec((FEAT, 1024), lambda i: (0, 0)),
            pl.BlockSpec((1, 1024), lambda i: (0, 0)),
            pl.BlockSpec((1024, 512), lambda i: (0, 0)),
        ],
        out_specs=pl.BlockSpec((4, ROWB, FEAT), lambda i: (0, i, 0)),
        out_shape=jax.ShapeDtypeStruct((4, N, FEAT), jnp.float32),
    )(agg1, xs0, dinv, Wg0, bg0.reshape(1, -1), Wg1)


def _tc_conv2_body(a_ref, zs_ref, dinv_ref, b1_ref, w2_ref, out_ref):
    dinv = dinv_ref[...]
    parts = [a_ref[0, j] + a_ref[1, j] + zs_ref[j] for j in range(4)]
    y = jnp.concatenate(parts, axis=1) * dinv
    h2 = jnp.maximum(y + b1_ref[...], 0.0)
    out_ref[...] = _dot(h2 * dinv, w2_ref[...])


def _tc_conv2(agg2, zs2c, dinv, bg1, Wg2):
    grid = N // ROWB
    return pl.pallas_call(
        _tc_conv2_body,
        grid=(grid,),
        in_specs=[
            pl.BlockSpec((NCORE, 4, ROWB, FEAT), lambda i: (0, 0, i, 0)),
            pl.BlockSpec((4, ROWB, FEAT), lambda i: (0, i, 0)),
            pl.BlockSpec((ROWB, 1), lambda i: (i, 0)),
            pl.BlockSpec((1, 512), lambda i: (0, 0)),
            pl.BlockSpec((512, FEAT), lambda i: (0, 0)),
        ],
        out_specs=pl.BlockSpec((ROWB, FEAT), lambda i: (i, 0)),
        out_shape=jax.ShapeDtypeStruct((N, FEAT), jnp.float32),
    )(agg2, zs2c, dinv, bg1.reshape(1, -1), Wg2)


def _tc_conv3_body(a_ref, zs_ref, dinv_ref, b2_ref, out_ref):
    y = (a_ref[0, 0] + a_ref[1, 0] + zs_ref[...]) * dinv_ref[...]
    out_ref[...] = jnp.maximum(y + b2_ref[...], 0.0)


def _tc_conv3(agg3, zs3, dinv, bg2):
    grid = N // ROWB
    return pl.pallas_call(
        _tc_conv3_body,
        grid=(grid,),
        in_specs=[
            pl.BlockSpec((NCORE, 1, ROWB, FEAT), lambda i: (0, 0, i, 0)),
            pl.BlockSpec((ROWB, FEAT), lambda i: (i, 0)),
            pl.BlockSpec((ROWB, 1), lambda i: (i, 0)),
            pl.BlockSpec((1, FEAT), lambda i: (0, 0)),
        ],
        out_specs=pl.BlockSpec((ROWB, FEAT), lambda i: (i, 0)),
        out_shape=jax.ShapeDtypeStruct((N, FEAT), jnp.float32),
    )(agg3, zs3, dinv, bg2.reshape(1, -1))


def _tc_head_body(x1_ref, x2_ref, x3_ref, wc0, bc0, wc1, bc1, wc2, bc2,
                  wf0, bf0, wf1, bf1, wf2, bf2, out_ref):
    t = jnp.maximum(_dot(x3_ref[...], wc0[...]) + bc0[...], 0.0)
    t = jnp.maximum(_dot(t, wc1[...]) + bc1[...], 0.0)
    t3 = _dot(t, wc2[...]) + bc2[...]
    x = jnp.concatenate([x1_ref[...], x2_ref[...], t3], axis=1)
    z = jnp.maximum(_dot(x, wf0[...]) + bf0[...], 0.0)
    z = jnp.maximum(_dot(z, wf1[...]) + bf1[...], 0.0)
    out_ref[...] = _dot(z, wf2[...]) + bf2[...]


def _tc_head(x1, x2, x3, Wc0, bc0, Wc1, bc1, Wc2, bc2,
             Wf0, bf0, Wf1, bf1, Wf2, bf2):
    grid = B // HEADB

    def full(shape):
        return pl.BlockSpec(shape, lambda i: tuple(0 for _ in shape))

    return pl.pallas_call(
        _tc_head_body,
        grid=(grid,),
        in_specs=[
            pl.BlockSpec((HEADB, FEAT), lambda i: (i, 0)),
            pl.BlockSpec((HEADB, FEAT), lambda i: (i, 0)),
            pl.BlockSpec((HEADB, CTX), lambda i: (i, 0)),
            full((CTX, 2048)), full((1, 2048)),
            full((2048, 512)), full((1, 512)),
            full((512, FEAT)), full((1, FEAT)),
            full((384, FEAT)), full((1, FEAT)),
            full((FEAT, 64)), full((1, 64)),
            full((64, 1)), full((1, 1)),
        ],
        out_specs=pl.BlockSpec((HEADB, 1), lambda i: (i, 0)),
        out_shape=jax.ShapeDtypeStruct((B, 1), jnp.float32),
    )(x1, x2, x3, Wc0, bc0.reshape(1, -1), Wc1, bc1.reshape(1, -1),
      Wc2, bc2.reshape(1, -1), Wf0, bf0.reshape(1, -1), Wf1, bf1.reshape(1, -1),
      Wf2, bf2.reshape(1, -1))


def kernel(inputs, node_feature, edge_index, Wg0, bg0, Wg1, bg1, Wg2, bg2,
           context_table, Wc0, bc0, Wc1, bc1, Wc2, bc2,
           Wf0, bf0, Wf1, bf1, Wf2, bf2):
    pad = E_PAD - E
    src = jnp.concatenate([edge_index[0], jnp.zeros((pad,), jnp.int32)])
    dst = jnp.concatenate([edge_index[1], jnp.full((pad,), N, jnp.int32)])
    ea = NSUB * NB0 * K
    srcA = src[:ea].reshape(NSUB, NB0, K)
    dstA = dst[:ea].reshape(NSUB, NB0, K)
    srcB = src[ea:].reshape(NSUB, NB1, K)
    dstB = dst[ea:].reshape(NSUB, NB1, K)
    zeros = jnp.zeros((N_ACC2, HW), jnp.float32)
    zeros16 = jnp.zeros((N_ACC, 16), jnp.float32)

    hist = _sc_degree(dstA, dstB, zeros16)
    dinv, xs0 = _tc_prep(hist, node_feature)

    agg1 = _sc_agg1(xs0.reshape(1, N, FEAT), srcA, dstA, srcB, dstB, zeros)
    zs2c = _tc_conv1(agg1, xs0, dinv, Wg0, bg0, Wg1)

    agg2 = _sc_agg4(zs2c, srcA, dstA, srcB, dstB, zeros)
    zs3 = _tc_conv2(agg2, zs2c, dinv, bg1, Wg2)

    agg3 = _sc_agg1(zs3.reshape(1, N, FEAT), srcA, dstA, srcB, dstB, zeros)
    h = _tc_conv3(agg3, zs3, dinv, bg2)

    i1 = inputs[:, 0].reshape(NW, K)
    i2 = inputs[:, 1].reshape(NW, K)
    i3 = inputs[:, 2].reshape(NW, K)
    x1, x2, x3 = _sc_pair(h, context_table, i1, i2, i3)

    return _tc_head(x1, x2, x3, Wc0, bc0, Wc1, bc1, Wc2, bc2,
                    Wf0, bf0, Wf1, bf1, Wf2, bf2)


# speedup vs baseline: 2.4105x; 1.0757x over previous
import functools

import jax
import jax.numpy as jnp
from jax import lax
from jax.experimental import pallas as pl
from jax.experimental.pallas import tpu as pltpu
from jax.experimental.pallas import tpu_sc as plsc

N = 10000
FEAT = 128
CTX = 288
CTXP = 384
B = 4096
E = 320000

NCORE = 2
NSUB = 16
NW = NCORE * NSUB
K = 128
NBC = 40
NB0 = 80
NB1 = 80
E_PAD = NSUB * (NB0 + NB1) * K
N_ACC = 10240
RPW = N_ACC // NSUB
HW = 64
N_ACC2 = 10016
RPW2 = N_ACC2 // NSUB
TRW = N // NSUB

ROWB = 400
HEADB = 512


def _mesh():
    return plsc.VectorSubcoreMesh(core_axis_name="c", subcore_axis_name="s")


@functools.partial(
    pl.kernel,
    out_type=jax.ShapeDtypeStruct((NCORE, N_ACC, 16), jnp.float32),
    mesh=_mesh(),
    compiler_params=pltpu.CompilerParams(use_tc_tiling_on_sc=False),
    scratch_types=[
        pltpu.VMEM((NBC, K), jnp.int32),
        pltpu.VMEM((K, 16), jnp.float32),
        pltpu.VMEM_SHARED((N_ACC, 16), jnp.float32),
    ],
)
def _sc_degree(dstA, dstB, zeros16, out, dstv, onesb, acc):
    c = lax.axis_index("c")
    s = lax.axis_index("s")
    for r in range(K):
        onesb[r] = jnp.ones((16,), jnp.float32)
    pltpu.sync_copy(zeros16.at[pl.ds(s * RPW, RPW)], acc.at[pl.ds(s * RPW, RPW)])
    plsc.subcore_barrier()

    def run(dsti, nb):
        @pl.loop(0, nb // NBC)
        def _(ch):
            pltpu.sync_copy(dsti.at[s, pl.ds(ch * NBC, NBC)], dstv)

            @pl.loop(0, NBC)
            def _(g):
                pltpu.sync_copy(onesb, acc.at[dstv.at[g]], add=True)

    @pl.when(c == 0)
    def _():
        run(dstA, NB0)

    @pl.when(c == 1)
    def _():
        run(dstB, NB1)

    plsc.subcore_barrier()
    pltpu.sync_copy(acc.at[pl.ds(s * RPW, RPW)], out.at[c, pl.ds(s * RPW, RPW)])


def _make_sc_agg(nj):
    @functools.partial(
        pl.kernel,
        out_type=jax.ShapeDtypeStruct((NCORE, nj, N_ACC2, FEAT), jnp.float32),
        mesh=_mesh(),
        compiler_params=pltpu.CompilerParams(use_tc_tiling_on_sc=False),
        scratch_types=[
            pltpu.VMEM((NBC, K), jnp.int32),
            pltpu.VMEM((NBC, K), jnp.int32),
            pltpu.VMEM((K, HW), jnp.float32),
            pltpu.VMEM((K, HW), jnp.float32),
            pltpu.VMEM((K, HW), jnp.float32),
            pltpu.VMEM((K, HW), jnp.float32),
            pltpu.VMEM_SHARED((N, HW), jnp.float32),
            pltpu.VMEM_SHARED((N_ACC2, HW), jnp.float32),
            pltpu.SemaphoreType.DMA,
            pltpu.SemaphoreType.DMA,
        ],
    )
    def agg(table, srcA, dstA, srcB, dstB, zeros, out,
            srcv, dstv, bufa, bufb, bufc, bufd, tbl, acc, semg, sems):
        c = lax.axis_index("c")
        s = lax.axis_index("s")
        bufs = (bufa, bufb, bufc, bufd)

        def wait_gather(b):
            pltpu.make_async_copy(tbl.at[pl.ds(0, K)], b, semg).wait()

        def wait_scatter():
            pltpu.make_async_copy(bufa, acc.at[dstv.at[0]], sems).wait()

        def run(srci, dsti, nb):
            @pl.loop(0, nb // NBC)
            def _(ch):
                pltpu.sync_copy(srci.at[s, pl.ds(ch * NBC, NBC)], srcv)
                pltpu.sync_copy(dsti.at[s, pl.ds(ch * NBC, NBC)], dstv)
                for g in range(3):
                    pltpu.async_copy(tbl.at[srcv.at[g]], bufs[g], semg)
                for g in range(NBC):
                    b = bufs[g % 4]
                    wait_gather(b)
                    pltpu.async_copy(b, acc.at[dstv.at[g]], sems, add=True)
                    if g + 3 < NBC:
                        if g >= 1:
                            wait_scatter()
                        pltpu.async_copy(tbl.at[srcv.at[g + 3]], bufs[(g + 3) % 4], semg)
                for _ in range(min(4, NBC)):
                    wait_scatter()

        for j in range(nj):
            for h in range(2):
                pltpu.sync_copy(
                    table.at[j, pl.ds(s * TRW, TRW), pl.ds(h * HW, HW)],
                    tbl.at[pl.ds(s * TRW, TRW)],
                )
                pltpu.sync_copy(
                    zeros.at[pl.ds(s * RPW2, RPW2)], acc.at[pl.ds(s * RPW2, RPW2)]
                )
                plsc.subcore_barrier()

                @pl.when(c == 0)
                def _():
                    run(srcA, dstA, NB0)

                @pl.when(c == 1)
                def _():
                    run(srcB, dstB, NB1)

                plsc.subcore_barrier()
                pltpu.sync_copy(
                    acc.at[pl.ds(s * RPW2, RPW2)],
                    out.at[c, j, pl.ds(s * RPW2, RPW2), pl.ds(h * HW, HW)],
                )
                if j + 1 < nj or h == 0:
                    plsc.subcore_barrier()

    return agg


_sc_agg1 = _make_sc_agg(1)
_sc_agg4 = _make_sc_agg(4)


@functools.partial(
    pl.kernel,
    out_type=[
        jax.ShapeDtypeStruct((B, FEAT), jnp.float32),
        jax.ShapeDtypeStruct((B, FEAT), jnp.float32),
        jax.ShapeDtypeStruct((B, CTXP), jnp.float32),
    ],
    mesh=_mesh(),
    compiler_params=pltpu.CompilerParams(use_tc_tiling_on_sc=False),
    scratch_types=[
        pltpu.VMEM((K,), jnp.int32),
        pltpu.VMEM((K, FEAT), jnp.float32),
        pltpu.VMEM((K, CTXP), jnp.float32),
        pltpu.SemaphoreType.DMA,
    ],
)
def _sc_pair(h, ctx, i1, i2, i3, o1, o2, o3, idxv, bufh, bufc, sem):
    c = lax.axis_index("c")
    s = lax.axis_index("s")
    w = s * NCORE + c
    base = w * K
    pltpu.sync_copy(i1.at[w], idxv)
    pltpu.async_copy(h.at[idxv], bufh, sem).wait()
    pltpu.sync_copy(bufh, o1.at[pl.ds(base, K)])
    pltpu.sync_copy(i2.at[w], idxv)
    pltpu.async_copy(h.at[idxv], bufh, sem).wait()
    pltpu.sync_copy(bufh, o2.at[pl.ds(base, K)])
    pltpu.sync_copy(i3.at[w], idxv)
    pltpu.async_copy(ctx.at[idxv], bufc, sem).wait()
    pltpu.sync_copy(bufc, o3.at[pl.ds(base, K)])


def _dot(a, b):
    return jnp.dot(a, b, preferred_element_type=jnp.float32)


def _tc_prep_body(hist_ref, nf_ref, dinv_ref, xs0_ref):
    h = hist_ref[...]
    deg = h[0, :, 0] + h[1, :, 0] + 1.0
    dinv = lax.rsqrt(deg).reshape(ROWB, 1)
    dinv_ref[...] = dinv
    xs0_ref[...] = nf_ref[...] * dinv


def _tc_prep(hist, node_feature):
    grid = N // ROWB
    return pl.pallas_call(
        _tc_prep_body,
        grid=(grid,),
        in_specs=[
            pl.BlockSpec((NCORE, ROWB, 16), lambda i: (0, i, 0)),
            pl.BlockSpec((ROWB, FEAT), lambda i: (i, 0)),
        ],
        out_specs=[
            pl.BlockSpec((ROWB, 1), lambda i: (i, 0)),
            pl.BlockSpec((ROWB, FEAT), lambda i: (i, 0)),
        ],
        out_shape=[
            jax.ShapeDtypeStruct((N, 1), jnp.float32),
            jax.ShapeDtypeStruct((N, FEAT), jnp.float32),
        ],
    )(hist, node_feature)


def _tc_conv1_body(a_ref, xs_ref, dinv_ref, w0_ref, b0_ref, w1_ref, out_ref):
    dinv = dinv_ref[...]
    y = (a_ref[0, 0] + a_ref[1, 0] + xs_ref[...]) * dinv
    h1 = jnp.maximum(_dot(y, w0_ref[...]) + b0_ref[...], 0.0)
    zs = _dot(h1 * dinv, w1_ref[...])
    for j in range(4):
        out_ref[j] = zs[:, j * FEAT:(j + 1) * FEAT]


def _tc_conv1(agg1, xs0, dinv, Wg0, bg0, Wg1):
    grid = N // ROWB
    return pl.pallas_call(
        _tc_conv1_body,
        grid=(grid,),
        in_specs=[
            pl.BlockSpec((NCORE, 1, ROWB, FEAT), lambda i: (0, 0, i, 0)),
            pl.BlockSpec((ROWB, FEAT), lambda i: (i, 0)),
            pl.BlockSpec((ROWB, 1), lambda i: (i, 0)),
            pl.BlockSpec((FEAT, 1024), lambda i: (0, 0)),
            pl.BlockSpec((1, 1024), lambda i: (0, 0)),
            pl.BlockSpec((1024, 512), lambda i: (0, 0)),
        ],
        out_specs=pl.BlockSpec((4, ROWB, FEAT), lambda i: (0, i, 0)),
        out_shape=jax.ShapeDtypeStruct((4, N, FEAT), jnp.float32),
    )(agg1, xs0, dinv, Wg0, bg0.reshape(1, -1), Wg1)


def _tc_conv2_body(a_ref, zs_ref, dinv_ref, b1_ref, w2_ref, out_ref):
    dinv = dinv_ref[...]
    parts = [a_ref[0, j] + a_ref[1, j] + zs_ref[j] for j in range(4)]
    y = jnp.concatenate(parts, axis=1) * dinv
    h2 = jnp.maximum(y + b1_ref[...], 0.0)
    out_ref[...] = _dot(h2 * dinv, w2_ref[...])


def _tc_conv2(agg2, zs2c, dinv, bg1, Wg2):
    grid = N // ROWB
    return pl.pallas_call(
        _tc_conv2_body,
        grid=(grid,),
        in_specs=[
            pl.BlockSpec((NCORE, 4, ROWB, FEAT), lambda i: (0, 0, i, 0)),
            pl.BlockSpec((4, ROWB, FEAT), lambda i: (0, i, 0)),
            pl.BlockSpec((ROWB, 1), lambda i: (i, 0)),
            pl.BlockSpec((1, 512), lambda i: (0, 0)),
            pl.BlockSpec((512, FEAT), lambda i: (0, 0)),
        ],
        out_specs=pl.BlockSpec((ROWB, FEAT), lambda i: (i, 0)),
        out_shape=jax.ShapeDtypeStruct((N, FEAT), jnp.float32),
    )(agg2, zs2c, dinv, bg1.reshape(1, -1), Wg2)


def _tc_conv3_body(a_ref, zs_ref, dinv_ref, b2_ref, out_ref):
    y = (a_ref[0, 0] + a_ref[1, 0] + zs_ref[...]) * dinv_ref[...]
    out_ref[...] = jnp.maximum(y + b2_ref[...], 0.0)


def _tc_conv3(agg3, zs3, dinv, bg2):
    grid = N // ROWB
    return pl.pallas_call(
        _tc_conv3_body,
        grid=(grid,),
        in_specs=[
            pl.BlockSpec((NCORE, 1, ROWB, FEAT), lambda i: (0, 0, i, 0)),
            pl.BlockSpec((ROWB, FEAT), lambda i: (i, 0)),
            pl.BlockSpec((ROWB, 1), lambda i: (i, 0)),
            pl.BlockSpec((1, FEAT), lambda i: (0, 0)),
        ],
        out_specs=pl.BlockSpec((ROWB, FEAT), lambda i: (i, 0)),
        out_shape=jax.ShapeDtypeStruct((N, FEAT), jnp.float32),
    )(agg3, zs3, dinv, bg2.reshape(1, -1))


def _tc_head_body(x1_ref, x2_ref, x3_ref, wc0, bc0, wc1, bc1, wc2, bc2,
                  wf0, bf0, wf1, bf1, wf2, bf2, out_ref):
    t = jnp.maximum(_dot(x3_ref[...], wc0[...]) + bc0[...], 0.0)
    t = jnp.maximum(_dot(t, wc1[...]) + bc1[...], 0.0)
    t3 = _dot(t, wc2[...]) + bc2[...]
    x = jnp.concatenate([x1_ref[...], x2_ref[...], t3], axis=1)
    z = jnp.maximum(_dot(x, wf0[...]) + bf0[...], 0.0)
    z = jnp.maximum(_dot(z, wf1[...]) + bf1[...], 0.0)
    out_ref[...] = _dot(z, wf2[...]) + bf2[...]


def _tc_head(x1, x2, x3, Wc0, bc0, Wc1, bc1, Wc2, bc2,
             Wf0, bf0, Wf1, bf1, Wf2, bf2):
    grid = B // HEADB

    def full(shape):
        return pl.BlockSpec(shape, lambda i: tuple(0 for _ in shape))

    return pl.pallas_call(
        _tc_head_body,
        grid=(grid,),
        in_specs=[
            pl.BlockSpec((HEADB, FEAT), lambda i: (i, 0)),
            pl.BlockSpec((HEADB, FEAT), lambda i: (i, 0)),
            pl.BlockSpec((HEADB, CTXP), lambda i: (i, 0)),
            full((CTXP, 2048)), full((1, 2048)),
            full((2048, 512)), full((1, 512)),
            full((512, FEAT)), full((1, FEAT)),
            full((384, FEAT)), full((1, FEAT)),
            full((FEAT, 64)), full((1, 64)),
            full((64, 1)), full((1, 1)),
        ],
        out_specs=pl.BlockSpec((HEADB, 1), lambda i: (i, 0)),
        out_shape=jax.ShapeDtypeStruct((B, 1), jnp.float32),
    )(x1, x2, x3, Wc0, bc0.reshape(1, -1), Wc1, bc1.reshape(1, -1),
      Wc2, bc2.reshape(1, -1), Wf0, bf0.reshape(1, -1), Wf1, bf1.reshape(1, -1),
      Wf2, bf2.reshape(1, -1))


def kernel(inputs, node_feature, edge_index, Wg0, bg0, Wg1, bg1, Wg2, bg2,
           context_table, Wc0, bc0, Wc1, bc1, Wc2, bc2,
           Wf0, bf0, Wf1, bf1, Wf2, bf2):
    pad = E_PAD - E
    src = jnp.concatenate([edge_index[0], jnp.zeros((pad,), jnp.int32)])
    dst = jnp.concatenate([edge_index[1], jnp.full((pad,), N, jnp.int32)])
    ea = NSUB * NB0 * K
    srcA = src[:ea].reshape(NSUB, NB0, K)
    dstA = dst[:ea].reshape(NSUB, NB0, K)
    srcB = src[ea:].reshape(NSUB, NB1, K)
    dstB = dst[ea:].reshape(NSUB, NB1, K)
    zeros = jnp.zeros((N_ACC2, HW), jnp.float32)
    zeros16 = jnp.zeros((N_ACC, 16), jnp.float32)

    hist = _sc_degree(dstA, dstB, zeros16)
    dinv, xs0 = _tc_prep(hist, node_feature)

    agg1 = _sc_agg1(xs0.reshape(1, N, FEAT), srcA, dstA, srcB, dstB, zeros)
    zs2c = _tc_conv1(agg1, xs0, dinv, Wg0, bg0, Wg1)

    agg2 = _sc_agg4(zs2c, srcA, dstA, srcB, dstB, zeros)
    zs3 = _tc_conv2(agg2, zs2c, dinv, bg1, Wg2)

    agg3 = _sc_agg1(zs3.reshape(1, N, FEAT), srcA, dstA, srcB, dstB, zeros)
    h = _tc_conv3(agg3, zs3, dinv, bg2)

    i1 = inputs[:, 0].reshape(NW, K)
    i2 = inputs[:, 1].reshape(NW, K)
    i3 = inputs[:, 2].reshape(NW, K)
    ctxp = jnp.pad(context_table, ((0, 0), (0, CTXP - CTX)))
    wc0p = jnp.pad(Wc0, ((0, CTXP - CTX), (0, 0)))
    x1, x2, x3 = _sc_pair(h, ctxp, i1, i2, i3)

    return _tc_head(x1, x2, x3, wc0p, bc0, Wc1, bc1, Wc2, bc2,
                    Wf0, bf0, Wf1, bf1, Wf2, bf2)
